# speedup vs baseline: 3.9408x; 3.9408x over previous
import functools

import numpy as np
import jax
import jax.numpy as jnp
from jax import lax
from jax.experimental import pallas as pl
from jax.experimental.pallas import tpu as pltpu
from jax.experimental.pallas import tpu_sc as plsc

VOCAB = 100000
EMBED = 128
MAXLEN = 512
B = 1024
L = 200

NUM_WORKERS = 32
ROWS_PER_W = B * L // NUM_WORKERS
CHUNK = L
N_CHUNKS = ROWS_PER_W // CHUNK
LANES = 16
GROUPS = EMBED // LANES


def _make_pe():
    pe = np.zeros((MAXLEN, EMBED), dtype=np.float32)
    position = np.arange(0, MAXLEN)[:, np.newaxis]
    div_term = np.exp(np.arange(0, EMBED, 2) * -(np.log(10000.0) / EMBED))
    pe[:, 0::2] = np.sin(position * div_term)
    pe[:, 1::2] = np.cos(position * div_term)
    return jnp.asarray(pe[:L, :])


_MESH = plsc.VectorSubcoreMesh(core_axis_name="c", subcore_axis_name="s")


@functools.partial(
    pl.kernel,
    mesh=_MESH,
    out_type=jax.ShapeDtypeStruct((B * L, EMBED), jnp.float32),
    scratch_types=[
        pltpu.VMEM((CHUNK,), jnp.int32),
        pltpu.VMEM((CHUNK, EMBED), jnp.float32),
        pltpu.VMEM((L, EMBED), jnp.float32),
        pltpu.SemaphoreType.DMA,
    ],
)
def _emb_lookup(x_hbm, w_hbm, pe_hbm, out_hbm, idx_v, rows_v, pe_v, sem):
    wid = lax.axis_index("s") * 2 + lax.axis_index("c")
    base = wid * ROWS_PER_W

    pltpu.sync_copy(pe_hbm, pe_v)

    def chunk_body(c, carry):
        off = base + c * CHUNK
        pltpu.sync_copy(x_hbm.at[pl.ds(off, CHUNK)], idx_v)
        pltpu.async_copy(w_hbm.at[idx_v], rows_v, sem).wait()

        def row_body(r, rcarry):
            for g in range(GROUPS):
                sl = pl.ds(g * LANES, LANES)
                rows_v[r, sl] = rows_v[r, sl] + pe_v[r, sl]
            return rcarry

        lax.fori_loop(0, CHUNK, row_body, 0)
        pltpu.sync_copy(rows_v, out_hbm.at[pl.ds(off, CHUNK)])
        return carry

    lax.fori_loop(0, N_CHUNKS, chunk_body, 0)


def kernel(x, W):
    pe = _make_pe()
    out = _emb_lookup(x.reshape(-1), W, pe)
    return out.reshape(B, L, EMBED)
